# baseline (speedup 1.0000x reference)
import functools
import jax
import jax.numpy as jnp
from jax import lax
from jax.experimental import pallas as pl
from jax.experimental.pallas import tpu as pltpu
from jax.experimental.pallas import tpu_sc as plsc

EMB = 16
NUM_F = 26
NC, NS = 2, 16
NW = NC * NS
IDX_PER_GATHER = 128


def _make_repack(nrows):
    n_chunks = (nrows + 127) // 128
    out_rows = n_chunks * 16
    per_w = (n_chunks + NW - 1) // NW
    mesh = plsc.VectorSubcoreMesh(core_axis_name="c", subcore_axis_name="s")

    @functools.partial(
        pl.kernel,
        mesh=mesh,
        out_type=jax.ShapeDtypeStruct((out_rows, 128), jnp.float32),
        scratch_types=[
            pltpu.VMEM((16, 128), jnp.float32),
            pltpu.VMEM((16, 128), jnp.float32),
            pltpu.VMEM((16, 128), jnp.float32),
            pltpu.VMEM((16, 128), jnp.float32),
            pltpu.SemaphoreType.DMA,
            pltpu.SemaphoreType.DMA,
            pltpu.SemaphoreType.DMA,
            pltpu.SemaphoreType.DMA,
        ],
        compiler_params=pltpu.CompilerParams(
            use_tc_tiling_on_sc=True, disable_bounds_checks=True,
            needs_layout_passes=False),
    )
    def repack_k(tabT_hbm, out_hbm, s0, s1, p0, p1, ss0, ss1, ws0, ws1):
        wid = lax.axis_index("s") * NC + lax.axis_index("c")
        c0 = wid * per_w
        c1 = jnp.minimum(c0 + per_w, n_chunks)
        npairs = (c1 - c0 + 1) // 2
        lane = lax.iota(jnp.int32, 16)
        cmax = n_chunks - 1

        def src_at(c):
            cc = pl.multiple_of(jnp.minimum(c, cmax) * 128, 128)
            return tabT_hbm.at[:, pl.ds(cc, 128)]

        def dst_at(c):
            cc = pl.multiple_of(jnp.minimum(c, cmax) * 16, 16)
            return out_hbm.at[pl.ds(cc, 16)]

        pltpu.async_copy(src_at(c0), s0, ss0)
        pltpu.async_copy(src_at(c0 + 1), s1, ss1)

        def half(i, c, s_v, p_v, ss, ws):
            pltpu.make_async_copy(src_at(c), s_v, ss).wait()

            @pl.when(i > 0)
            def _drain():
                pltpu.make_async_copy(p_v, dst_at(c), ws).wait()

            for m in range(8):
                xs = []
                for d in range(16):
                    cv = ((lane + d) & 15) + 16 * m
                    xs.append((cv, plsc.load_gather(s_v, [lane, cv])))
                for cv, x in xs:
                    plsc.store_scatter(
                        p_v,
                        [lax.shift_right_logical(cv, 3),
                         lax.shift_left(cv & 7, 4) + lane], x)
            pltpu.async_copy(src_at(c + 2), s_v, ss)
            pltpu.async_copy(p_v, dst_at(c), ws)

        def pair(i, carry):
            c = c0 + 2 * i
            half(i, c, s0, p0, ss0, ws0)
            half(i, c + 1, s1, p1, ss1, ws1)
            return carry

        lax.fori_loop(0, npairs, pair, 0)
        pltpu.make_async_copy(src_at(c0), s0, ss0).wait()
        pltpu.make_async_copy(src_at(c0), s1, ss1).wait()
        pltpu.make_async_copy(p0, dst_at(c0), ws0).wait()
        pltpu.make_async_copy(p1, dst_at(c0), ws1).wait()

    return repack_k


def _make_gather(batch):
    spw = batch // NW
    SPG = 16
    npg = spw // SPG
    mesh = plsc.VectorSubcoreMesh(core_axis_name="c", subcore_axis_name="s")

    @functools.partial(
        pl.kernel,
        mesh=mesh,
        out_type=jax.ShapeDtypeStruct((batch * NUM_F, EMB), jnp.float32),
        scratch_types=[
            pltpu.VMEM((spw, 128), jnp.int32),
            pltpu.VMEM((spw, NUM_F), jnp.int32),
            pltpu.VMEM((SPG * NUM_F, EMB), jnp.float32),
            pltpu.SemaphoreType.DMA,
        ],
        compiler_params=pltpu.CompilerParams(use_tc_tiling_on_sc=False),
    )
    def gather_k(x1_hbm, table_hbm, out_hbm, idx_v, idxp_v, rows_v, sem):
        wid = lax.axis_index("s") * NC + lax.axis_index("c")
        s0 = wid * spw
        pltpu.sync_copy(x1_hbm.at[pl.ds(s0, spw)], idx_v)

        def repack(s, carry):
            idxp_v[s, pl.ds(0, 16)] = idx_v[s, pl.ds(0, 16)]
            idxp_v[s, pl.ds(NUM_F - 16, 16)] = idx_v[s, pl.ds(NUM_F - 16, 16)]
            return carry

        lax.fori_loop(0, spw, repack, 0)

        def body(g, carry):
            copies = []
            for t in range(SPG):
                copies.append(pltpu.async_copy(
                    table_hbm.at[idxp_v.at[g * SPG + t]],
                    rows_v.at[pl.ds(t * NUM_F, NUM_F)], sem))
            for c in copies:
                c.wait()
            pltpu.sync_copy(
                rows_v,
                out_hbm.at[pl.ds((s0 + g * SPG) * NUM_F, SPG * NUM_F)])
            return carry

        lax.fori_loop(0, npg, body, 0)

    return gather_k


def _mlp_body(flat_ref, x2_ref, W1_ref, b1_ref, W2_ref, b2_ref, Wh_ref,
              Wx_ref, bfx_ref, S_ref, out_ref):
    flat = flat_ref[...]
    h = jnp.dot(flat, W1_ref[...], preferred_element_type=jnp.float32)
    h = jnp.maximum(h + b1_ref[...], 0.0)
    h = jnp.dot(h, W2_ref[...], preferred_element_type=jnp.float32)
    h = jnp.maximum(h + b2_ref[...], 0.0)
    z = (jnp.dot(h, Wh_ref[...], preferred_element_type=jnp.float32)
         + jnp.dot(x2_ref[...], Wx_ref[...], preferred_element_type=jnp.float32)
         + bfx_ref[...])
    z = jnp.maximum(z, 0.0)
    s = jnp.dot(flat, S_ref[...], preferred_element_type=jnp.float32)
    fm = (jnp.sum(s * s, axis=1, keepdims=True)
          - jnp.sum(flat * flat, axis=1, keepdims=True))
    out_ref[...] = jax.nn.sigmoid(z + 0.5 * fm)


def _mlp_call(flat, x2, W1, b1, W2, b2, Wh, Wx, bfx, S, block_b=512):
    b, in_dim = flat.shape
    grid = (b // block_b,)
    full = lambda shape: pl.BlockSpec(shape, lambda i: (0, 0))
    return pl.pallas_call(
        _mlp_body,
        grid=grid,
        in_specs=[
            pl.BlockSpec((block_b, in_dim), lambda i: (i, 0)),
            pl.BlockSpec((block_b, x2.shape[1]), lambda i: (i, 0)),
            full(W1.shape), full(b1.shape), full(W2.shape), full(b2.shape),
            full(Wh.shape), full(Wx.shape), full(bfx.shape), full(S.shape),
        ],
        out_specs=pl.BlockSpec((block_b, 1), lambda i: (i, 0)),
        out_shape=jax.ShapeDtypeStruct((b, 1), jnp.float32),
    )(flat, x2, W1, b1, W2, b2, Wh, Wx, bfx, S)


def kernel(x1, x2, table, W1, b1, W2, b2, Wfx, bfx):
    b = x1.shape[0]
    x1p = jnp.pad(x1.astype(jnp.int32), ((0, 0), (0, 128 - NUM_F)))
    nrows = table.shape[0]
    tpack = _make_repack(nrows)(table.T)
    tview = tpack.reshape(tpack.shape[0] * 8, EMB)
    rows = _make_gather(b)(x1p, tview)
    flat = rows.reshape(b, NUM_F * EMB)

    S = jnp.tile(jnp.eye(EMB, dtype=jnp.float32), (NUM_F, 1))
    out = _mlp_call(
        flat, x2, W1, b1.reshape(1, -1), W2, b2.reshape(1, -1),
        Wfx[:W2.shape[1]], Wfx[W2.shape[1]:], bfx.reshape(1, 1), S)
    return out.reshape(-1)


# speedup vs baseline: 1.0528x; 1.0528x over previous
import functools
import jax
import jax.numpy as jnp
from jax import lax
from jax.experimental import pallas as pl
from jax.experimental.pallas import tpu as pltpu
from jax.experimental.pallas import tpu_sc as plsc

EMB = 16
NUM_F = 26
NC, NS = 2, 16
NW = NC * NS
IDX_PER_GATHER = 128


def _make_repack(nrows):
    n_chunks = (nrows + 127) // 128
    out_rows = n_chunks * 16
    per_w = (n_chunks + NW - 1) // NW
    mesh = plsc.VectorSubcoreMesh(core_axis_name="c", subcore_axis_name="s")

    @functools.partial(
        pl.kernel,
        mesh=mesh,
        out_type=jax.ShapeDtypeStruct((out_rows, 128), jnp.float32),
        scratch_types=[
            pltpu.VMEM((16, 128), jnp.float32),
            pltpu.VMEM((16, 128), jnp.float32),
            pltpu.VMEM((16, 128), jnp.float32),
            pltpu.VMEM((16, 128), jnp.float32),
            pltpu.SemaphoreType.DMA,
            pltpu.SemaphoreType.DMA,
            pltpu.SemaphoreType.DMA,
            pltpu.SemaphoreType.DMA,
        ],
        compiler_params=pltpu.CompilerParams(
            use_tc_tiling_on_sc=True, disable_bounds_checks=True,
            needs_layout_passes=False),
    )
    def repack_k(tabT_hbm, out_hbm, s0, s1, p0, p1, ss0, ss1, ws0, ws1):
        wid = lax.axis_index("s") * NC + lax.axis_index("c")
        c0 = wid * per_w
        c1 = jnp.minimum(c0 + per_w, n_chunks)
        npairs = (c1 - c0 + 1) // 2
        lane = lax.iota(jnp.int32, 16)
        cmax = n_chunks - 1

        def src_at(c):
            cc = pl.multiple_of(jnp.minimum(c, cmax) * 128, 128)
            return tabT_hbm.at[:, pl.ds(cc, 128)]

        def dst_at(c):
            cc = pl.multiple_of(jnp.minimum(c, cmax) * 16, 16)
            return out_hbm.at[pl.ds(cc, 16)]

        pltpu.async_copy(src_at(c0), s0, ss0)
        pltpu.async_copy(src_at(c0 + 1), s1, ss1)

        def half(i, c, s_v, p_v, ss, ws):
            pltpu.make_async_copy(src_at(c), s_v, ss).wait()

            @pl.when(i > 0)
            def _drain():
                pltpu.make_async_copy(p_v, dst_at(c), ws).wait()

            for m in range(8):
                xs = []
                for d in range(16):
                    cv = ((lane + d) & 15) + 16 * m
                    xs.append((cv, plsc.load_gather(s_v, [lane, cv])))
                for cv, x in xs:
                    plsc.store_scatter(
                        p_v,
                        [lax.shift_right_logical(cv, 3),
                         lax.shift_left(cv & 7, 4) + lane], x)
            pltpu.async_copy(src_at(c + 2), s_v, ss)
            pltpu.async_copy(p_v, dst_at(c), ws)

        def pair(i, carry):
            c = c0 + 2 * i
            half(i, c, s0, p0, ss0, ws0)
            half(i, c + 1, s1, p1, ss1, ws1)
            return carry

        lax.fori_loop(0, npairs, pair, 0)
        pltpu.make_async_copy(src_at(c0), s0, ss0).wait()
        pltpu.make_async_copy(src_at(c0), s1, ss1).wait()
        pltpu.make_async_copy(p0, dst_at(c0), ws0).wait()
        pltpu.make_async_copy(p1, dst_at(c0), ws1).wait()

    return repack_k


def _make_gather(batch, base):
    spw = batch // NW
    SPG = 16
    npg = spw // SPG
    mesh = plsc.VectorSubcoreMesh(core_axis_name="c", subcore_axis_name="s")

    @functools.partial(
        pl.kernel,
        mesh=mesh,
        out_type=jax.ShapeDtypeStruct((batch * NUM_F, EMB), jnp.float32),
        scratch_types=[
            pltpu.VMEM((spw, 128), jnp.int32),
            pltpu.VMEM((spw, NUM_F), jnp.int32),
            pltpu.VMEM((SPG * NUM_F, EMB), jnp.float32),
            pltpu.SemaphoreType.DMA,
        ],
        compiler_params=pltpu.CompilerParams(use_tc_tiling_on_sc=False),
    )
    def gather_k(x1_hbm, table_hbm, out_hbm, idx_v, idxp_v, rows_v, sem):
        wid = lax.axis_index("s") * NC + lax.axis_index("c")
        s0 = wid * spw
        pltpu.sync_copy(x1_hbm.at[pl.ds(base + s0, spw)], idx_v)

        def repack(s, carry):
            idxp_v[s, pl.ds(0, 16)] = idx_v[s, pl.ds(0, 16)]
            idxp_v[s, pl.ds(NUM_F - 16, 16)] = idx_v[s, pl.ds(NUM_F - 16, 16)]
            return carry

        lax.fori_loop(0, spw, repack, 0)

        def body(g, carry):
            copies = []
            for t in range(SPG):
                copies.append(pltpu.async_copy(
                    table_hbm.at[idxp_v.at[g * SPG + t]],
                    rows_v.at[pl.ds(t * NUM_F, NUM_F)], sem))
            for c in copies:
                c.wait()
            pltpu.sync_copy(
                rows_v,
                out_hbm.at[pl.ds((s0 + g * SPG) * NUM_F, SPG * NUM_F)])
            return carry

        lax.fori_loop(0, npg, body, 0)

    return gather_k


def _mlp_body(flat_ref, x2_ref, W1_ref, b1_ref, W2_ref, b2_ref, Wh_ref,
              Wx_ref, bfx_ref, S_ref, out_ref):
    flat = flat_ref[...]
    h = jnp.dot(flat, W1_ref[...], preferred_element_type=jnp.float32)
    h = jnp.maximum(h + b1_ref[...], 0.0)
    h = jnp.dot(h, W2_ref[...], preferred_element_type=jnp.float32)
    h = jnp.maximum(h + b2_ref[...], 0.0)
    z = (jnp.dot(h, Wh_ref[...], preferred_element_type=jnp.float32)
         + jnp.dot(x2_ref[...], Wx_ref[...], preferred_element_type=jnp.float32)
         + bfx_ref[...])
    z = jnp.maximum(z, 0.0)
    s = jnp.dot(flat, S_ref[...], preferred_element_type=jnp.float32)
    fm = (jnp.sum(s * s, axis=1, keepdims=True)
          - jnp.sum(flat * flat, axis=1, keepdims=True))
    out_ref[...] = jax.nn.sigmoid(z + 0.5 * fm)


def _mlp_call(flat, x2, W1, b1, W2, b2, Wh, Wx, bfx, S, base=0, block_b=512):
    b, in_dim = flat.shape
    grid = (b // block_b,)
    off = base // block_b
    full = lambda shape: pl.BlockSpec(shape, lambda i: (0, 0))
    return pl.pallas_call(
        _mlp_body,
        grid=grid,
        in_specs=[
            pl.BlockSpec((block_b, in_dim), lambda i: (i, 0)),
            pl.BlockSpec((block_b, x2.shape[1]), lambda i: (i + off, 0)),
            full(W1.shape), full(b1.shape), full(W2.shape), full(b2.shape),
            full(Wh.shape), full(Wx.shape), full(bfx.shape), full(S.shape),
        ],
        out_specs=pl.BlockSpec((block_b, 1), lambda i: (i, 0)),
        out_shape=jax.ShapeDtypeStruct((b, 1), jnp.float32),
    )(flat, x2, W1, b1, W2, b2, Wh, Wx, bfx, S)


def kernel(x1, x2, table, W1, b1, W2, b2, Wfx, bfx):
    b = x1.shape[0]
    x1p = jnp.pad(x1.astype(jnp.int32), ((0, 0), (0, 128 - NUM_F)))
    nrows = table.shape[0]
    tpack = _make_repack(nrows)(table.T)
    tview = tpack.reshape(tpack.shape[0] * 8, EMB)
    S = jnp.tile(jnp.eye(EMB, dtype=jnp.float32), (NUM_F, 1))
    NCHUNK = 2
    sb = b // NCHUNK
    outs = []
    for ci in range(NCHUNK):
        rows = _make_gather(sb, ci * sb)(x1p, tview)
        flat = rows.reshape(sb, NUM_F * EMB)
        outs.append(_mlp_call(
            flat, x2, W1, b1.reshape(1, -1), W2, b2.reshape(1, -1),
            Wfx[:W2.shape[1]], Wfx[W2.shape[1]:], bfx.reshape(1, 1), S,
            base=ci * sb))
    return jnp.concatenate(outs, axis=0).reshape(-1)


# speedup vs baseline: 1.0547x; 1.0017x over previous
import functools
import jax
import jax.numpy as jnp
from jax import lax
from jax.experimental import pallas as pl
from jax.experimental.pallas import tpu as pltpu
from jax.experimental.pallas import tpu_sc as plsc

EMB = 16
NUM_F = 26
NC, NS = 2, 16
NW = NC * NS
IDX_PER_GATHER = 128


def _make_repack(nrows):
    n_chunks = (nrows + 127) // 128
    out_rows = n_chunks * 16
    per_w = (n_chunks + NW - 1) // NW
    mesh = plsc.VectorSubcoreMesh(core_axis_name="c", subcore_axis_name="s")

    @functools.partial(
        pl.kernel,
        mesh=mesh,
        out_type=jax.ShapeDtypeStruct((out_rows, 128), jnp.float32),
        scratch_types=[
            pltpu.VMEM((16, 128), jnp.float32),
            pltpu.VMEM((16, 128), jnp.float32),
            pltpu.VMEM((16, 128), jnp.float32),
            pltpu.VMEM((16, 128), jnp.float32),
            pltpu.SemaphoreType.DMA,
            pltpu.SemaphoreType.DMA,
            pltpu.SemaphoreType.DMA,
            pltpu.SemaphoreType.DMA,
        ],
        compiler_params=pltpu.CompilerParams(
            use_tc_tiling_on_sc=True, disable_bounds_checks=True,
            needs_layout_passes=False),
    )
    def repack_k(tabT_hbm, out_hbm, s0, s1, p0, p1, ss0, ss1, ws0, ws1):
        wid = lax.axis_index("s") * NC + lax.axis_index("c")
        c0 = wid * per_w
        c1 = jnp.minimum(c0 + per_w, n_chunks)
        npairs = (c1 - c0 + 1) // 2
        lane = lax.iota(jnp.int32, 16)
        cmax = n_chunks - 1

        def src_at(c):
            cc = pl.multiple_of(jnp.minimum(c, cmax) * 128, 128)
            return tabT_hbm.at[:, pl.ds(cc, 128)]

        def dst_at(c):
            cc = pl.multiple_of(jnp.minimum(c, cmax) * 16, 16)
            return out_hbm.at[pl.ds(cc, 16)]

        pltpu.async_copy(src_at(c0), s0, ss0)
        pltpu.async_copy(src_at(c0 + 1), s1, ss1)

        def half(i, c, s_v, p_v, ss, ws):
            pltpu.make_async_copy(src_at(c), s_v, ss).wait()

            @pl.when(i > 0)
            def _drain():
                pltpu.make_async_copy(p_v, dst_at(c), ws).wait()

            for m in range(8):
                xs = []
                for d in range(16):
                    cv = ((lane + d) & 15) + 16 * m
                    xs.append((cv, plsc.load_gather(s_v, [lane, cv])))
                for cv, x in xs:
                    plsc.store_scatter(
                        p_v,
                        [lax.shift_right_logical(cv, 3),
                         lax.shift_left(cv & 7, 4) + lane], x)
            pltpu.async_copy(src_at(c + 2), s_v, ss)
            pltpu.async_copy(p_v, dst_at(c), ws)

        def pair(i, carry):
            c = c0 + 2 * i
            half(i, c, s0, p0, ss0, ws0)
            half(i, c + 1, s1, p1, ss1, ws1)
            return carry

        lax.fori_loop(0, npairs, pair, 0)
        pltpu.make_async_copy(src_at(c0), s0, ss0).wait()
        pltpu.make_async_copy(src_at(c0), s1, ss1).wait()
        pltpu.make_async_copy(p0, dst_at(c0), ws0).wait()
        pltpu.make_async_copy(p1, dst_at(c0), ws1).wait()

    return repack_k


def _make_gather(batch, base):
    spw = batch // NW
    SPG = 16
    npg = spw // SPG
    mesh = plsc.VectorSubcoreMesh(core_axis_name="c", subcore_axis_name="s")

    @functools.partial(
        pl.kernel,
        mesh=mesh,
        out_type=jax.ShapeDtypeStruct((batch * NUM_F, EMB), jnp.float32),
        scratch_types=[
            pltpu.VMEM((spw, 128), jnp.int32),
            pltpu.VMEM((spw, NUM_F), jnp.int32),
            pltpu.VMEM((SPG * NUM_F, EMB), jnp.float32),
            pltpu.SemaphoreType.DMA,
        ],
        compiler_params=pltpu.CompilerParams(use_tc_tiling_on_sc=False),
    )
    def gather_k(x1_hbm, table_hbm, out_hbm, idx_v, idxp_v, rows_v, sem):
        wid = lax.axis_index("s") * NC + lax.axis_index("c")
        s0 = wid * spw
        pltpu.sync_copy(x1_hbm.at[pl.ds(base + s0, spw)], idx_v)

        def repack(s, carry):
            idxp_v[s, pl.ds(0, 16)] = idx_v[s, pl.ds(0, 16)]
            idxp_v[s, pl.ds(NUM_F - 16, 16)] = idx_v[s, pl.ds(NUM_F - 16, 16)]
            return carry

        lax.fori_loop(0, spw, repack, 0)

        def body(g, carry):
            copies = []
            for t in range(SPG):
                copies.append(pltpu.async_copy(
                    table_hbm.at[idxp_v.at[g * SPG + t]],
                    rows_v.at[pl.ds(t * NUM_F, NUM_F)], sem))
            for c in copies:
                c.wait()
            pltpu.sync_copy(
                rows_v,
                out_hbm.at[pl.ds((s0 + g * SPG) * NUM_F, SPG * NUM_F)])
            return carry

        lax.fori_loop(0, npg, body, 0)

    return gather_k


def _mlp_body(flat_ref, x2_ref, W1_ref, b1_ref, W2_ref, b2_ref, Wh_ref,
              Wx_ref, bfx_ref, S_ref, out_ref):
    flat = flat_ref[...]
    h = jnp.dot(flat, W1_ref[...], preferred_element_type=jnp.float32)
    h = jnp.maximum(h + b1_ref[...], 0.0)
    h = jnp.dot(h, W2_ref[...], preferred_element_type=jnp.float32)
    h = jnp.maximum(h + b2_ref[...], 0.0)
    z = (jnp.dot(h, Wh_ref[...], preferred_element_type=jnp.float32)
         + jnp.dot(x2_ref[...], Wx_ref[...], preferred_element_type=jnp.float32)
         + bfx_ref[...])
    z = jnp.maximum(z, 0.0)
    s = jnp.dot(flat, S_ref[...], preferred_element_type=jnp.float32)
    fm = (jnp.sum(s * s, axis=1, keepdims=True)
          - jnp.sum(flat * flat, axis=1, keepdims=True))
    out_ref[...] = jax.nn.sigmoid(z + 0.5 * fm)


def _mlp_call(flat, x2, W1, b1, W2, b2, Wh, Wx, bfx, S, base=0, block_b=512):
    b, in_dim = flat.shape
    grid = (b // block_b,)
    off = base // block_b
    full = lambda shape: pl.BlockSpec(shape, lambda i: (0, 0))
    return pl.pallas_call(
        _mlp_body,
        grid=grid,
        in_specs=[
            pl.BlockSpec((block_b, in_dim), lambda i: (i, 0)),
            pl.BlockSpec((block_b, x2.shape[1]), lambda i: (i + off, 0)),
            full(W1.shape), full(b1.shape), full(W2.shape), full(b2.shape),
            full(Wh.shape), full(Wx.shape), full(bfx.shape), full(S.shape),
        ],
        out_specs=pl.BlockSpec((block_b, 1), lambda i: (i, 0)),
        out_shape=jax.ShapeDtypeStruct((b, 1), jnp.float32),
    )(flat, x2, W1, b1, W2, b2, Wh, Wx, bfx, S)


def kernel(x1, x2, table, W1, b1, W2, b2, Wfx, bfx):
    b = x1.shape[0]
    x1p = jnp.pad(x1.astype(jnp.int32), ((0, 0), (0, 128 - NUM_F)))
    nrows = table.shape[0]
    tpack = _make_repack(nrows)(table.T)
    tview = tpack.reshape(tpack.shape[0] * 8, EMB)
    S = jnp.tile(jnp.eye(EMB, dtype=jnp.float32), (NUM_F, 1))
    NCHUNK = 4
    sb = b // NCHUNK
    outs = []
    for ci in range(NCHUNK):
        rows = _make_gather(sb, ci * sb)(x1p, tview)
        flat = rows.reshape(sb, NUM_F * EMB)
        outs.append(_mlp_call(
            flat, x2, W1, b1.reshape(1, -1), W2, b2.reshape(1, -1),
            Wfx[:W2.shape[1]], Wfx[W2.shape[1]:], bfx.reshape(1, 1), S,
            base=ci * sb))
    return jnp.concatenate(outs, axis=0).reshape(-1)


# speedup vs baseline: 1.0636x; 1.0085x over previous
import functools
import jax
import jax.numpy as jnp
from jax import lax
from jax.experimental import pallas as pl
from jax.experimental.pallas import tpu as pltpu
from jax.experimental.pallas import tpu_sc as plsc

EMB = 16
NUM_F = 26
NC, NS = 2, 16
NW = NC * NS
IDX_PER_GATHER = 128


def _make_repack(nrows):
    n_chunks = (nrows + 255) // 256
    out_rows = n_chunks * 32
    per_w = (n_chunks + NW - 1) // NW
    mesh = plsc.VectorSubcoreMesh(core_axis_name="c", subcore_axis_name="s")

    @functools.partial(
        pl.kernel,
        mesh=mesh,
        out_type=jax.ShapeDtypeStruct((out_rows, 128), jnp.float32),
        scratch_types=[
            pltpu.VMEM((16, 256), jnp.float32),
            pltpu.VMEM((16, 256), jnp.float32),
            pltpu.VMEM((32, 128), jnp.float32),
            pltpu.VMEM((32, 128), jnp.float32),
            pltpu.SemaphoreType.DMA,
            pltpu.SemaphoreType.DMA,
            pltpu.SemaphoreType.DMA,
            pltpu.SemaphoreType.DMA,
        ],
        compiler_params=pltpu.CompilerParams(
            use_tc_tiling_on_sc=True, disable_bounds_checks=True,
            needs_layout_passes=False),
    )
    def repack_k(tabT_hbm, out_hbm, s0, s1, p0, p1, ss0, ss1, ws0, ws1):
        wid = lax.axis_index("s") * NC + lax.axis_index("c")
        c0 = wid * per_w
        c1 = jnp.minimum(c0 + per_w, n_chunks)
        npairs = (c1 - c0 + 1) // 2
        lane = lax.iota(jnp.int32, 16)
        cmax = n_chunks - 1

        def src_at(c):
            cc = pl.multiple_of(jnp.minimum(c, cmax) * 256, 128)
            return tabT_hbm.at[:, pl.ds(cc, 256)]

        def dst_at(c):
            cc = pl.multiple_of(jnp.minimum(c, cmax) * 32, 16)
            return out_hbm.at[pl.ds(cc, 32)]

        pltpu.async_copy(src_at(c0), s0, ss0)
        pltpu.async_copy(src_at(c0 + 1), s1, ss1)

        def half(i, c, s_v, p_v, ss, ws):
            pltpu.make_async_copy(src_at(c), s_v, ss).wait()

            @pl.when(i > 0)
            def _drain():
                pltpu.make_async_copy(p_v, dst_at(c), ws).wait()

            for m in range(16):
                xs = []
                for d in range(16):
                    cv = ((lane + d) & 15) + 16 * m
                    xs.append((cv, plsc.load_gather(s_v, [lane, cv])))
                for cv, x in xs:
                    plsc.store_scatter(
                        p_v,
                        [lax.shift_right_logical(cv, 3),
                         lax.shift_left(cv & 7, 4) + lane], x)
            pltpu.async_copy(src_at(c + 2), s_v, ss)
            pltpu.async_copy(p_v, dst_at(c), ws)

        def pair(i, carry):
            c = c0 + 2 * i
            half(i, c, s0, p0, ss0, ws0)
            half(i, c + 1, s1, p1, ss1, ws1)
            return carry

        lax.fori_loop(0, npairs, pair, 0)
        pltpu.make_async_copy(src_at(c0), s0, ss0).wait()
        pltpu.make_async_copy(src_at(c0), s1, ss1).wait()
        pltpu.make_async_copy(p0, dst_at(c0), ws0).wait()
        pltpu.make_async_copy(p1, dst_at(c0), ws1).wait()

    return repack_k


def _make_gather(batch, base):
    spw = batch // NW
    SPG = 16
    npg = spw // SPG
    mesh = plsc.VectorSubcoreMesh(core_axis_name="c", subcore_axis_name="s")

    @functools.partial(
        pl.kernel,
        mesh=mesh,
        out_type=jax.ShapeDtypeStruct((batch * NUM_F, EMB), jnp.float32),
        scratch_types=[
            pltpu.VMEM((spw, 128), jnp.int32),
            pltpu.VMEM((spw, NUM_F), jnp.int32),
            pltpu.VMEM((SPG * NUM_F, EMB), jnp.float32),
            pltpu.SemaphoreType.DMA,
        ],
        compiler_params=pltpu.CompilerParams(use_tc_tiling_on_sc=False),
    )
    def gather_k(x1_hbm, table_hbm, out_hbm, idx_v, idxp_v, rows_v, sem):
        wid = lax.axis_index("s") * NC + lax.axis_index("c")
        s0 = wid * spw
        pltpu.sync_copy(x1_hbm.at[pl.ds(base + s0, spw)], idx_v)

        def repack(s, carry):
            idxp_v[s, pl.ds(0, 16)] = idx_v[s, pl.ds(0, 16)]
            idxp_v[s, pl.ds(NUM_F - 16, 16)] = idx_v[s, pl.ds(NUM_F - 16, 16)]
            return carry

        lax.fori_loop(0, spw, repack, 0)

        def body(g, carry):
            copies = []
            for t in range(SPG):
                copies.append(pltpu.async_copy(
                    table_hbm.at[idxp_v.at[g * SPG + t]],
                    rows_v.at[pl.ds(t * NUM_F, NUM_F)], sem))
            for c in copies:
                c.wait()
            pltpu.sync_copy(
                rows_v,
                out_hbm.at[pl.ds((s0 + g * SPG) * NUM_F, SPG * NUM_F)])
            return carry

        lax.fori_loop(0, npg, body, 0)

    return gather_k


def _mlp_body(flat_ref, x2_ref, W1_ref, b1_ref, W2_ref, b2_ref, Wh_ref,
              Wx_ref, bfx_ref, S_ref, out_ref):
    flat = flat_ref[...]
    h = jnp.dot(flat, W1_ref[...], preferred_element_type=jnp.float32)
    h = jnp.maximum(h + b1_ref[...], 0.0)
    h = jnp.dot(h, W2_ref[...], preferred_element_type=jnp.float32)
    h = jnp.maximum(h + b2_ref[...], 0.0)
    z = (jnp.dot(h, Wh_ref[...], preferred_element_type=jnp.float32)
         + jnp.dot(x2_ref[...], Wx_ref[...], preferred_element_type=jnp.float32)
         + bfx_ref[...])
    z = jnp.maximum(z, 0.0)
    s = jnp.dot(flat, S_ref[...], preferred_element_type=jnp.float32)
    fm = (jnp.sum(s * s, axis=1, keepdims=True)
          - jnp.sum(flat * flat, axis=1, keepdims=True))
    out_ref[...] = jax.nn.sigmoid(z + 0.5 * fm)


def _mlp_call(flat, x2, W1, b1, W2, b2, Wh, Wx, bfx, S, base=0, block_b=512):
    b, in_dim = flat.shape
    grid = (b // block_b,)
    off = base // block_b
    full = lambda shape: pl.BlockSpec(shape, lambda i: (0, 0))
    return pl.pallas_call(
        _mlp_body,
        grid=grid,
        in_specs=[
            pl.BlockSpec((block_b, in_dim), lambda i: (i, 0)),
            pl.BlockSpec((block_b, x2.shape[1]), lambda i: (i + off, 0)),
            full(W1.shape), full(b1.shape), full(W2.shape), full(b2.shape),
            full(Wh.shape), full(Wx.shape), full(bfx.shape), full(S.shape),
        ],
        out_specs=pl.BlockSpec((block_b, 1), lambda i: (i, 0)),
        out_shape=jax.ShapeDtypeStruct((b, 1), jnp.float32),
    )(flat, x2, W1, b1, W2, b2, Wh, Wx, bfx, S)


def kernel(x1, x2, table, W1, b1, W2, b2, Wfx, bfx):
    b = x1.shape[0]
    x1p = jnp.pad(x1.astype(jnp.int32), ((0, 0), (0, 128 - NUM_F)))
    nrows = table.shape[0]
    tpack = _make_repack(nrows)(table.T)
    tview = tpack.reshape(tpack.shape[0] * 8, EMB)
    S = jnp.tile(jnp.eye(EMB, dtype=jnp.float32), (NUM_F, 1))
    NCHUNK = 4
    sb = b // NCHUNK
    outs = []
    for ci in range(NCHUNK):
        rows = _make_gather(sb, ci * sb)(x1p, tview)
        flat = rows.reshape(sb, NUM_F * EMB)
        outs.append(_mlp_call(
            flat, x2, W1, b1.reshape(1, -1), W2, b2.reshape(1, -1),
            Wfx[:W2.shape[1]], Wfx[W2.shape[1]:], bfx.reshape(1, 1), S,
            base=ci * sb))
    return jnp.concatenate(outs, axis=0).reshape(-1)
